# Initial kernel scaffold; baseline (speedup 1.0000x reference)
import jax
import jax.numpy as jnp
from jax.experimental import pallas as pl


def kernel(tokens, router_w, router_b, expert_weights):
    raise NotImplementedError("write your pallas kernel here")


# speedup vs baseline: 2.8233x; 2.8233x over previous
import jax
import jax.numpy as jnp
from jax.experimental import pallas as pl


def _moe_body(x_ref, rw_ref, rb_ref, w_ref, o_ref):
    m, d = x_ref.shape
    e = rw_ref.shape[1]
    xb = x_ref[...]
    logits = jnp.dot(xb, rw_ref[...], preferred_element_type=jnp.float32)
    logits = logits + rb_ref[...]
    p = jax.nn.softmax(logits, axis=-1)
    iota = jax.lax.broadcasted_iota(jnp.int32, (m, e), 1)
    m1 = jnp.max(p, axis=-1, keepdims=True)
    i1 = jnp.min(jnp.where(p == m1, iota, e), axis=-1, keepdims=True)
    pm = jnp.where(iota == i1, -jnp.inf, p)
    m2 = jnp.max(pm, axis=-1, keepdims=True)
    i2 = jnp.min(jnp.where(pm == m2, iota, e), axis=-1, keepdims=True)
    comb = jnp.where((iota == i1) | (iota == i2), p, 0.0) / (m1 + m2)
    acc = jnp.zeros((m, d), jnp.float32)
    for ei in range(e):
        y = jax.lax.dot_general(
            xb, w_ref[ei], (((1,), (1,)), ((), ())),
            preferred_element_type=jnp.float32)
        acc = acc + comb[:, ei:ei + 1] * y
    o_ref[...] = acc


def kernel(tokens, router_w, router_b, expert_weights):
    b, s, d = tokens.shape
    e = router_w.shape[1]
    x = tokens.reshape(b * s, d)
    n = b * s
    M = 256
    out = pl.pallas_call(
        _moe_body,
        grid=(n // M,),
        in_specs=[
            pl.BlockSpec((M, d), lambda i: (i, 0)),
            pl.BlockSpec((d, e), lambda i: (0, 0)),
            pl.BlockSpec((1, e), lambda i: (0, 0)),
            pl.BlockSpec((e, d, d), lambda i: (0, 0, 0)),
        ],
        out_specs=pl.BlockSpec((M, d), lambda i: (i, 0)),
        out_shape=jax.ShapeDtypeStruct((n, d), jnp.float32),
    )(x, router_w, router_b.reshape(1, e), expert_weights)
    return out.reshape(b, s, d)
